# baseline (speedup 1.0000x reference)
import jax
import jax.numpy as jnp
from jax import lax
from jax.experimental import pallas as pl
from jax.experimental.pallas import tpu as pltpu
from jax.experimental.pallas import tpu_sc as plsc

_NC = 2
_NS = 16
_NW = _NC * _NS


def _transpose_table(lookup):
    V, D = lookup.shape
    PR = 128 // D
    W = 512
    VA = (V // W) * W
    n_win = VA // W
    lookup_t = jnp.transpose(lookup)
    n_slots = -(-n_win // _NW)
    n_slots += n_slots % 2
    mesh = plsc.VectorSubcoreMesh(core_axis_name="core", subcore_axis_name="subcore")

    import dataclasses
    cp = pltpu.CompilerParams()
    if "needs_layout_passes" in pltpu.CompilerParams.__dataclass_fields__:
        cp = dataclasses.replace(cp, needs_layout_passes=False)

    @pl.kernel(
        out_type=jax.ShapeDtypeStruct((V * D // 128, 128), lookup.dtype),
        mesh=mesh,
        compiler_params=cp,
        scratch_types=(
            [pltpu.VMEM((2, D, W), lookup.dtype),
             pltpu.VMEM((2, W // PR, 128), lookup.dtype)]
            + [pltpu.SemaphoreType.DMA] * 4
        ),
    )
    def transpose_kernel(tab_hbm, out_hbm, tbuf, obuf, *sems):
        isem = sems[:2]
        osem = sems[2:]
        wid = lax.axis_index("subcore") * _NC + lax.axis_index("core")
        c_lo = lax.broadcasted_iota(jnp.int32, (16,), 0)
        c_hi = c_lo + 16

        def win(k):
            return wid + _NW * k

        def i_copy(k, p):
            off = pl.multiple_of(win(k) * W, 128)
            return pltpu.make_async_copy(
                tab_hbm.at[:, pl.ds(off, W)], tbuf.at[p], isem[p])

        def o_copy(k, p):
            off = pl.multiple_of(win(k) * (W // PR), 8)
            return pltpu.make_async_copy(
                obuf.at[p], out_hbm.at[pl.ds(off, W // PR)], osem[p])

        for p in range(2):
            @pl.when(win(p) < n_win)
            def _():
                i_copy(p, p).start()

        @pl.loop(0, n_slots, step=2)
        def _(ki):
            for p in range(2):
                k = ki + p
                live = win(k) < n_win

                @pl.when(live)
                def _():
                    i_copy(k, p).wait()

                    @pl.when(k >= 2)
                    def _():
                        o_copy(k - 2, p).wait()

                    @pl.loop(0, W)
                    def _(t):
                        col = c_lo * 0 + t
                        line = t >> 2
                        word = (t & (PR - 1)) * D
                        obuf[p, line, pl.ds(word, 16)] = plsc.load_gather(
                            tbuf.at[p], [c_lo, col])
                        obuf[p, line, pl.ds(word + 16, 16)] = plsc.load_gather(
                            tbuf.at[p], [c_hi, col])

                    o_copy(k, p).start()

                @pl.when(win(k + 2) < n_win)
                def _():
                    i_copy(k + 2, p).start()

        for p in range(2):
            @pl.when(win(n_slots - 2 + p) < n_win)
            def _():
                o_copy(n_slots - 2 + p, p).wait()

    tabc = transpose_kernel(lookup_t)
    if VA < V:
        tail = lookup[VA:, :].reshape((V - VA) * D // 128, 128)
        tabc = lax.dynamic_update_slice(tabc, tail, (VA * D // 128, 0))
    return tabc


def _gather(token_indices, table_lin):
    B, S = token_indices.shape
    N = B * S
    D = table_lin.shape[1]
    b_per_w = N // _NW
    NBUF = 8
    C = S
    n_chunks = b_per_w // C
    assert n_chunks % NBUF == 0 and n_chunks >= NBUF

    idx = token_indices.reshape(N).astype(jnp.int32)
    mesh = plsc.VectorSubcoreMesh(core_axis_name="core", subcore_axis_name="subcore")

    @pl.kernel(
        out_type=jax.ShapeDtypeStruct((B, S, D), table_lin.dtype),
        mesh=mesh,
        compiler_params=pltpu.CompilerParams(use_tc_tiling_on_sc=False),
        scratch_types=(
            [pltpu.VMEM((b_per_w,), jnp.int32),
             pltpu.VMEM((NBUF, C, D), table_lin.dtype)]
            + [pltpu.SemaphoreType.DMA] * (1 + 2 * NBUF)
        ),
    )
    def gather_kernel(table_hbm, idx_hbm, out3_hbm, idx_v, rows_v, isem, *sems):
        gsem = sems[:NBUF]
        osem = sems[NBUF:]
        wid = lax.axis_index("subcore") * _NC + lax.axis_index("core")
        base = wid * b_per_w
        pltpu.async_copy(idx_hbm.at[pl.ds(base, b_per_w)], idx_v, isem).wait()

        def g_copy(g, b):
            return pltpu.make_async_copy(
                table_hbm.at[idx_v.at[pl.ds(g * C, C)]], rows_v.at[b], gsem[b])

        def o_copy(g, b):
            return pltpu.make_async_copy(
                rows_v.at[b], out3_hbm.at[(base + g * C) // S], osem[b])

        for b in range(NBUF):
            g_copy(b, b).start()

        @pl.loop(0, n_chunks, step=NBUF)
        def _(gi):
            for b in range(NBUF):
                g = gi + b
                g_copy(g, b).wait()
                o_copy(g, b).start()
                nxt = g + NBUF

                @pl.when(nxt < n_chunks)
                def _():
                    o_copy(g, b).wait()
                    g_copy(nxt, b).start()

        for b in range(NBUF):
            o_copy(n_chunks - NBUF + b, b).wait()

    return gather_kernel(table_lin, idx)


def kernel(token_indices, lookup):
    if token_indices.ndim == 1:
        token_indices = token_indices[None, :]
    V, D = lookup.shape
    tabc = _transpose_table(lookup)
    table_lin = tabc.reshape(V, D)
    return _gather(token_indices, table_lin)


# speedup vs baseline: 1.0002x; 1.0002x over previous
import jax
import jax.numpy as jnp
from jax import lax
from jax.experimental import pallas as pl
from jax.experimental.pallas import tpu as pltpu
from jax.experimental.pallas import tpu_sc as plsc

_NC = 2
_NS = 16
_NW = _NC * _NS


def _transpose_table(lookup):
    V, D = lookup.shape
    PR = 128 // D
    W = 512
    VA = (V // W) * W
    n_win = VA // W
    lookup_t = jnp.transpose(lookup)
    n_slots = -(-n_win // _NW)
    n_slots += n_slots % 2
    mesh = plsc.VectorSubcoreMesh(core_axis_name="core", subcore_axis_name="subcore")

    import dataclasses
    cp = pltpu.CompilerParams()
    if "needs_layout_passes" in pltpu.CompilerParams.__dataclass_fields__:
        cp = dataclasses.replace(cp, needs_layout_passes=False)

    @pl.kernel(
        out_type=jax.ShapeDtypeStruct((V * D // 128, 128), lookup.dtype),
        mesh=mesh,
        compiler_params=cp,
        scratch_types=(
            [pltpu.VMEM((2, D, W), lookup.dtype),
             pltpu.VMEM((2, W // PR, 128), lookup.dtype)]
            + [pltpu.SemaphoreType.DMA] * 4
        ),
    )
    def transpose_kernel(tab_hbm, out_hbm, tbuf, obuf, *sems):
        isem = sems[:2]
        osem = sems[2:]
        wid = lax.axis_index("subcore") * _NC + lax.axis_index("core")
        c_lo = lax.broadcasted_iota(jnp.int32, (16,), 0)
        c_hi = c_lo + 16

        def win(k):
            return wid + _NW * k

        def i_copy(k, p):
            off = pl.multiple_of(win(k) * W, 128)
            return pltpu.make_async_copy(
                tab_hbm.at[:, pl.ds(off, W)], tbuf.at[p], isem[p])

        def o_copy(k, p):
            off = pl.multiple_of(win(k) * (W // PR), 8)
            return pltpu.make_async_copy(
                obuf.at[p], out_hbm.at[pl.ds(off, W // PR)], osem[p])

        for p in range(2):
            @pl.when(win(p) < n_win)
            def _():
                i_copy(p, p).start()

        @pl.loop(0, n_slots, step=2)
        def _(ki):
            for p in range(2):
                k = ki + p
                live = win(k) < n_win

                @pl.when(live)
                def _():
                    i_copy(k, p).wait()

                    @pl.when(k >= 2)
                    def _():
                        o_copy(k - 2, p).wait()

                    @pl.loop(0, W)
                    def _(t):
                        col = c_lo * 0 + t
                        line = t >> 2
                        word = (t & (PR - 1)) * D
                        obuf[p, line, pl.ds(word, 16)] = plsc.load_gather(
                            tbuf.at[p], [c_lo, col])
                        obuf[p, line, pl.ds(word + 16, 16)] = plsc.load_gather(
                            tbuf.at[p], [c_hi, col])

                    o_copy(k, p).start()

                @pl.when(win(k + 2) < n_win)
                def _():
                    i_copy(k + 2, p).start()

        for k in range(max(0, n_slots - 4), n_slots):
            @pl.when((win(k) < n_win) & (win(k + 2) >= n_win))
            def _():
                o_copy(k, k % 2).wait()

    tabc = transpose_kernel(lookup_t)
    if VA < V:
        tail = lookup[VA:, :].reshape((V - VA) * D // 128, 128)
        tabc = lax.dynamic_update_slice(tabc, tail, (VA * D // 128, 0))
    return tabc


def _gather(token_indices, table_lin):
    B, S = token_indices.shape
    N = B * S
    D = table_lin.shape[1]
    b_per_w = N // _NW
    NBUF = 8
    C = S
    n_chunks = b_per_w // C
    assert n_chunks % NBUF == 0 and n_chunks >= NBUF

    idx = token_indices.reshape(N).astype(jnp.int32)
    mesh = plsc.VectorSubcoreMesh(core_axis_name="core", subcore_axis_name="subcore")

    @pl.kernel(
        out_type=jax.ShapeDtypeStruct((B, S, D), table_lin.dtype),
        mesh=mesh,
        compiler_params=pltpu.CompilerParams(use_tc_tiling_on_sc=False),
        scratch_types=(
            [pltpu.VMEM((b_per_w,), jnp.int32),
             pltpu.VMEM((NBUF, C, D), table_lin.dtype)]
            + [pltpu.SemaphoreType.DMA] * (1 + 2 * NBUF)
        ),
    )
    def gather_kernel(table_hbm, idx_hbm, out3_hbm, idx_v, rows_v, isem, *sems):
        gsem = sems[:NBUF]
        osem = sems[NBUF:]
        wid = lax.axis_index("subcore") * _NC + lax.axis_index("core")
        base = wid * b_per_w
        pltpu.async_copy(idx_hbm.at[pl.ds(base, b_per_w)], idx_v, isem).wait()

        def g_copy(g, b):
            return pltpu.make_async_copy(
                table_hbm.at[idx_v.at[pl.ds(g * C, C)]], rows_v.at[b], gsem[b])

        def o_copy(g, b):
            return pltpu.make_async_copy(
                rows_v.at[b], out3_hbm.at[(base + g * C) // S], osem[b])

        for b in range(NBUF):
            g_copy(b, b).start()

        @pl.loop(0, n_chunks, step=NBUF)
        def _(gi):
            for b in range(NBUF):
                g = gi + b
                g_copy(g, b).wait()
                o_copy(g, b).start()
                nxt = g + NBUF

                @pl.when(nxt < n_chunks)
                def _():
                    o_copy(g, b).wait()
                    g_copy(nxt, b).start()

        for b in range(NBUF):
            o_copy(n_chunks - NBUF + b, b).wait()

    return gather_kernel(table_lin, idx)


def kernel(token_indices, lookup):
    if token_indices.ndim == 1:
        token_indices = token_indices[None, :]
    V, D = lookup.shape
    tabc = _transpose_table(lookup)
    table_lin = tabc.reshape(V, D)
    return _gather(token_indices, table_lin)


# speedup vs baseline: 1.0025x; 1.0024x over previous
import jax
import jax.numpy as jnp
from jax import lax
from jax.experimental import pallas as pl
from jax.experimental.pallas import tpu as pltpu
from jax.experimental.pallas import tpu_sc as plsc

_NC = 2
_NS = 16
_NW = _NC * _NS


def _transpose_table(lookup):
    V, D = lookup.shape
    PR = 128 // D
    W = 512
    VA = (V // W) * W
    n_win = VA // W
    lookup_t = jnp.transpose(lookup)
    n_slots = -(-n_win // _NW)
    n_slots += n_slots % 2
    mesh = plsc.VectorSubcoreMesh(core_axis_name="core", subcore_axis_name="subcore")

    import dataclasses
    cp = pltpu.CompilerParams()
    if "needs_layout_passes" in pltpu.CompilerParams.__dataclass_fields__:
        cp = dataclasses.replace(cp, needs_layout_passes=False)

    @pl.kernel(
        out_type=jax.ShapeDtypeStruct((V * D // 128, 128), lookup.dtype),
        mesh=mesh,
        compiler_params=cp,
        scratch_types=(
            [pltpu.VMEM((2, D, W), lookup.dtype),
             pltpu.VMEM((2, W // PR, 128), lookup.dtype)]
            + [pltpu.SemaphoreType.DMA] * 4
        ),
    )
    def transpose_kernel(tab_hbm, out_hbm, tbuf, obuf, *sems):
        isem = sems[:2]
        osem = sems[2:]
        wid = lax.axis_index("subcore") * _NC + lax.axis_index("core")
        c_lo = lax.broadcasted_iota(jnp.int32, (16,), 0)
        c_hi = c_lo + 16

        def win(k):
            return wid + _NW * k

        def i_copy(k, p):
            off = pl.multiple_of(win(k) * W, 128)
            return pltpu.make_async_copy(
                tab_hbm.at[:, pl.ds(off, W)], tbuf.at[p], isem[p])

        def o_copy(k, p):
            off = pl.multiple_of(win(k) * (W // PR), 8)
            return pltpu.make_async_copy(
                obuf.at[p], out_hbm.at[pl.ds(off, W // PR)], osem[p])

        for p in range(2):
            @pl.when(win(p) < n_win)
            def _():
                i_copy(p, p).start()

        @pl.loop(0, n_slots, step=2)
        def _(ki):
            for p in range(2):
                k = ki + p
                live = win(k) < n_win

                @pl.when(live)
                def _():
                    i_copy(k, p).wait()

                    @pl.when(k >= 2)
                    def _():
                        o_copy(k - 2, p).wait()

                    @pl.loop(0, W, step=PR)
                    def _(t):
                        line = t >> 2
                        for u in range(PR):
                            col = c_lo * 0 + (t + u)
                            obuf[p, line, pl.ds(u * D, 16)] = (
                                plsc.load_gather(tbuf.at[p], [c_lo, col]))
                            obuf[p, line, pl.ds(u * D + 16, 16)] = (
                                plsc.load_gather(tbuf.at[p], [c_hi, col]))

                    o_copy(k, p).start()

                @pl.when(win(k + 2) < n_win)
                def _():
                    i_copy(k + 2, p).start()

        for k in range(max(0, n_slots - 4), n_slots):
            @pl.when((win(k) < n_win) & (win(k + 2) >= n_win))
            def _():
                o_copy(k, k % 2).wait()

    tabc = transpose_kernel(lookup_t)
    if VA < V:
        tail = lookup[VA:, :].reshape((V - VA) * D // 128, 128)
        tabc = lax.dynamic_update_slice(tabc, tail, (VA * D // 128, 0))
    return tabc


def _gather(token_indices, table_lin):
    B, S = token_indices.shape
    N = B * S
    D = table_lin.shape[1]
    b_per_w = N // _NW
    NBUF = 8
    C = S
    n_chunks = b_per_w // C
    assert n_chunks % NBUF == 0 and n_chunks >= NBUF

    idx = token_indices.reshape(N).astype(jnp.int32)
    mesh = plsc.VectorSubcoreMesh(core_axis_name="core", subcore_axis_name="subcore")

    @pl.kernel(
        out_type=jax.ShapeDtypeStruct((B, S, D), table_lin.dtype),
        mesh=mesh,
        compiler_params=pltpu.CompilerParams(use_tc_tiling_on_sc=False),
        scratch_types=(
            [pltpu.VMEM((b_per_w,), jnp.int32),
             pltpu.VMEM((NBUF, C, D), table_lin.dtype)]
            + [pltpu.SemaphoreType.DMA] * (1 + 2 * NBUF)
        ),
    )
    def gather_kernel(table_hbm, idx_hbm, out3_hbm, idx_v, rows_v, isem, *sems):
        gsem = sems[:NBUF]
        osem = sems[NBUF:]
        wid = lax.axis_index("subcore") * _NC + lax.axis_index("core")
        base = wid * b_per_w
        pltpu.async_copy(idx_hbm.at[pl.ds(base, b_per_w)], idx_v, isem).wait()

        def g_copy(g, b):
            return pltpu.make_async_copy(
                table_hbm.at[idx_v.at[pl.ds(g * C, C)]], rows_v.at[b], gsem[b])

        def o_copy(g, b):
            return pltpu.make_async_copy(
                rows_v.at[b], out3_hbm.at[(base + g * C) // S], osem[b])

        for b in range(NBUF):
            g_copy(b, b).start()

        @pl.loop(0, n_chunks, step=NBUF)
        def _(gi):
            for b in range(NBUF):
                g = gi + b
                g_copy(g, b).wait()
                o_copy(g, b).start()
                nxt = g + NBUF

                @pl.when(nxt < n_chunks)
                def _():
                    o_copy(g, b).wait()
                    g_copy(nxt, b).start()

        for b in range(NBUF):
            o_copy(n_chunks - NBUF + b, b).wait()

    return gather_kernel(table_lin, idx)


def kernel(token_indices, lookup):
    if token_indices.ndim == 1:
        token_indices = token_indices[None, :]
    V, D = lookup.shape
    tabc = _transpose_table(lookup)
    table_lin = tabc.reshape(V, D)
    return _gather(token_indices, table_lin)


# speedup vs baseline: 1.2818x; 1.2785x over previous
import jax
import jax.numpy as jnp
from jax import lax
from jax.experimental import pallas as pl
from jax.experimental.pallas import tpu as pltpu
from jax.experimental.pallas import tpu_sc as plsc

_NC = 2
_NS = 16
_NW = _NC * _NS


def kernel(token_indices, lookup):
    if token_indices.ndim == 1:
        token_indices = token_indices[None, :]
    B, S = token_indices.shape
    V, D = lookup.shape
    N = B * S
    assert N % _NW == 0
    b_per_w = N // _NW
    NBUF = 8
    C = S
    n_chunks = b_per_w // C
    assert n_chunks % NBUF == 0 and n_chunks >= NBUF

    idx = token_indices.reshape(N).astype(jnp.int32)
    mesh = plsc.VectorSubcoreMesh(core_axis_name="core", subcore_axis_name="subcore")

    @pl.kernel(
        out_type=jax.ShapeDtypeStruct((B, S, D), lookup.dtype),
        mesh=mesh,
        compiler_params=pltpu.CompilerParams(use_tc_tiling_on_sc=False),
        scratch_types=(
            [pltpu.VMEM((b_per_w,), jnp.int32),
             pltpu.VMEM((NBUF, C, D), lookup.dtype)]
            + [pltpu.SemaphoreType.DMA] * (1 + 2 * NBUF)
        ),
    )
    def gather_kernel(table_hbm, idx_hbm, out3_hbm, idx_v, rows_v, isem, *sems):
        gsem = sems[:NBUF]
        osem = sems[NBUF:]
        wid = lax.axis_index("subcore") * _NC + lax.axis_index("core")
        base = wid * b_per_w
        pltpu.async_copy(idx_hbm.at[pl.ds(base, b_per_w)], idx_v, isem).wait()

        def g_copy(g, b):
            return pltpu.make_async_copy(
                table_hbm.at[idx_v.at[pl.ds(g * C, C)]], rows_v.at[b], gsem[b])

        def o_copy(g, b):
            return pltpu.make_async_copy(
                rows_v.at[b], out3_hbm.at[(base + g * C) // S], osem[b])

        for b in range(NBUF):
            g_copy(b, b).start()

        @pl.loop(0, n_chunks, step=NBUF)
        def _(gi):
            for b in range(NBUF):
                g = gi + b
                g_copy(g, b).wait()
                o_copy(g, b).start()
                nxt = g + NBUF

                @pl.when(nxt < n_chunks)
                def _():
                    o_copy(g, b).wait()
                    g_copy(nxt, b).start()

        for b in range(NBUF):
            o_copy(n_chunks - NBUF + b, b).wait()

    return gather_kernel(lookup, idx)
